# Initial kernel scaffold; baseline (speedup 1.0000x reference)
import jax
import jax.numpy as jnp
from jax.experimental import pallas as pl


def kernel(x):
    raise NotImplementedError("write your pallas kernel here")


# speedup vs baseline: 32.4805x; 32.4805x over previous
import jax
import jax.numpy as jnp
from jax.experimental import pallas as pl

_N = 8192
_RANK = _N // 2 - 1
_ROWS = 64 * 32
_INT_MIN = -(2**31)


def _select_body(x_ref, o_ref):
    x = x_ref[...]
    v = jax.lax.bitcast_convert_type(x, jnp.int32)
    int_min = jnp.int32(_INT_MIN)
    skey = jnp.where(v >= 0, v, int_min - v)
    rows = x.shape[0]
    acc = jnp.zeros((rows, 1), jnp.int32)
    for bit in range(31, -1, -1):
        mask_val = jnp.int32(_INT_MIN if bit == 31 else 1 << bit)
        cand = acc | mask_val
        thr = cand ^ int_min
        cnt = jnp.sum((skey < thr).astype(jnp.int32), axis=1, keepdims=True)
        acc = jnp.where(cnt <= _RANK, cand, acc)
    skey_ans = acc ^ int_min
    vbits = jnp.where(skey_ans >= 0, skey_ans, int_min - skey_ans)
    o_ref[...] = jax.lax.bitcast_convert_type(vbits, jnp.float32)


def kernel(x):
    b0, b1, n = x.shape
    xr = x.reshape(b0 * b1, n)
    rows = b0 * b1
    block_rows = min(128, rows)
    out = pl.pallas_call(
        _select_body,
        grid=(rows // block_rows,),
        in_specs=[pl.BlockSpec((block_rows, n), lambda i: (i, 0))],
        out_specs=pl.BlockSpec((block_rows, 1), lambda i: (i, 0)),
        out_shape=jax.ShapeDtypeStruct((rows, 1), jnp.float32),
    )(xr)
    return out.reshape(b0, b1)
